# baseline (speedup 1.0000x reference)
import functools
import math

import jax
import jax.numpy as jnp
from jax import lax
from jax.experimental import pallas as pl
from jax.experimental.pallas import tpu as pltpu
from jax.experimental.pallas import tpu_sc as plsc

N_NODES = 10000
N_EDGES = 320000
DIMS = 128
NC = 2
NS = 16
NW = NC * NS
CHUNK = 128
NCHUNKS = 80
EPW = NCHUNKS * CHUNK
E_PAD = NW * EPW
N_ACC = 10240
NB = 2
DEG_WIN = 16
RPT = 624
RPT_LAST = N_NODES - 15 * RPT
BN_SCALE = 1.0 / math.sqrt(1.0 + 1e-5)

_mesh = plsc.VectorSubcoreMesh(core_axis_name="c", subcore_axis_name="s")


@functools.partial(
    pl.kernel,
    mesh=_mesh,
    out_type=jax.ShapeDtypeStruct((NC * N_NODES,), jnp.float32),
    scratch_types=[
        pltpu.VMEM((NCHUNKS, CHUNK), jnp.int32),
        pltpu.VMEM((CHUNK,), jnp.float32),
        pltpu.VMEM((N_ACC,), jnp.float32),
        pltpu.VMEM_SHARED((N_ACC,), jnp.float32),
        pltpu.SemaphoreType.DMA,
    ],
)
def _deg_kernel(dst_hbm, zeros_hbm, deg_out, dst_all, ones_v, stage_v,
                deg_sh, sem):
    c = lax.axis_index("c")
    s = lax.axis_index("s")
    w = c * NS + s
    pltpu.sync_copy(dst_hbm.at[pl.ds(w * NCHUNKS, NCHUNKS)], dst_all)
    for j in range(CHUNK // 16):
        ones_v[pl.ds(j * 16, 16)] = jnp.full((16,), 1.0, dtype=jnp.float32)

    @pl.when(s == 0)
    def _init():
        pltpu.sync_copy(zeros_hbm, stage_v)
        pltpu.sync_copy(stage_v, deg_sh)

    plsc.subcore_barrier()

    def body(i, carry):
        @pl.when(i >= DEG_WIN)
        def _throttle():
            pltpu.make_async_copy(ones_v, deg_sh.at[dst_all.at[0]], sem).wait()

        pltpu.async_copy(ones_v, deg_sh.at[dst_all.at[i]], sem, add=True)
        return carry

    lax.fori_loop(0, NCHUNKS, body, 0)

    def drain(i, carry):
        pltpu.make_async_copy(ones_v, deg_sh.at[dst_all.at[0]], sem).wait()
        return carry

    lax.fori_loop(0, DEG_WIN, drain, 0)
    plsc.subcore_barrier()

    @pl.when(s == 0)
    def _drain():
        pltpu.sync_copy(deg_sh.at[pl.ds(0, N_NODES)], stage_v.at[pl.ds(0, N_NODES)])
        pltpu.sync_copy(stage_v.at[pl.ds(0, N_NODES)],
                        deg_out.at[pl.ds(c * N_NODES, N_NODES)])


@functools.partial(
    pl.kernel,
    mesh=_mesh,
    out_type=jax.ShapeDtypeStruct((NC * N_NODES, DIMS), jnp.float32),
    scratch_types=[
        pltpu.VMEM((NCHUNKS, CHUNK), jnp.int32),
        pltpu.VMEM((NB, CHUNK), jnp.int32),
        pltpu.VMEM((NB, CHUNK, DIMS), jnp.float32),
        pltpu.VMEM_SHARED((N_ACC, DIMS), jnp.float32),
        pltpu.SemaphoreType.DMA,
        pltpu.SemaphoreType.DMA,
        pltpu.SemaphoreType.DMA,
        pltpu.SemaphoreType.DMA,
        pltpu.SemaphoreType.DMA,
        pltpu.SemaphoreType.DMA,
    ],
)
def _scatter_kernel(src_hbm, dst_hbm, g_hbm, zrows_hbm, acc_out,
                    src_all, dst_r, rows_v, acc_sh,
                    sem_g0, sem_g1, sem_d0, sem_d1, sem_s0, sem_s1):
    c = lax.axis_index("c")
    s = lax.axis_index("s")
    w = c * NS + s
    pltpu.sync_copy(src_hbm.at[pl.ds(w * NCHUNKS, NCHUNKS)], src_all)

    @pl.when(s < 15)
    def _init_a():
        pltpu.sync_copy(zrows_hbm.at[pl.ds(0, RPT)],
                        acc_sh.at[pl.ds(s * RPT, RPT)])

    @pl.when(s == 15)
    def _init_b():
        pltpu.sync_copy(zrows_hbm, acc_sh.at[pl.ds(15 * RPT, RPT_LAST)])

    plsc.subcore_barrier()

    pltpu.async_copy(g_hbm.at[src_all.at[0]], rows_v.at[0], sem_g0)
    pltpu.async_copy(dst_hbm.at[w * NCHUNKS], dst_r.at[0], sem_d0)
    pltpu.async_copy(g_hbm.at[src_all.at[1]], rows_v.at[1], sem_g1)
    pltpu.async_copy(dst_hbm.at[w * NCHUNKS + 1], dst_r.at[1], sem_d1)

    def body(k, carry):
        i0 = 2 * k
        i1 = i0 + 1
        pltpu.make_async_copy(dst_hbm.at[0], dst_r.at[0], sem_d0).wait()
        pltpu.make_async_copy(g_hbm.at[src_all.at[0]], rows_v.at[0],
                              sem_g0).wait()
        pltpu.async_copy(rows_v.at[0], acc_sh.at[dst_r.at[0]], sem_s0,
                         add=True)
        pltpu.make_async_copy(dst_hbm.at[0], dst_r.at[1], sem_d1).wait()
        pltpu.make_async_copy(g_hbm.at[src_all.at[1]], rows_v.at[1],
                              sem_g1).wait()
        pltpu.async_copy(rows_v.at[1], acc_sh.at[dst_r.at[1]], sem_s1,
                         add=True)
        pltpu.make_async_copy(rows_v.at[0], acc_sh.at[dst_r.at[0]],
                              sem_s0).wait()

        @pl.when(i0 + 2 < NCHUNKS)
        def _refill0():
            pltpu.async_copy(g_hbm.at[src_all.at[i0 + 2]], rows_v.at[0],
                             sem_g0)
            pltpu.async_copy(dst_hbm.at[w * NCHUNKS + i0 + 2], dst_r.at[0],
                             sem_d0)

        pltpu.make_async_copy(rows_v.at[1], acc_sh.at[dst_r.at[1]],
                              sem_s1).wait()

        @pl.when(i1 + 2 < NCHUNKS)
        def _refill1():
            pltpu.async_copy(g_hbm.at[src_all.at[i1 + 2]], rows_v.at[1],
                             sem_g1)
            pltpu.async_copy(dst_hbm.at[w * NCHUNKS + i1 + 2], dst_r.at[1],
                             sem_d1)

        return carry

    lax.fori_loop(0, NCHUNKS // 2, body, 0)
    plsc.subcore_barrier()

    @pl.when(s < 15)
    def _drain_a():
        pltpu.sync_copy(acc_sh.at[pl.ds(s * RPT, RPT)],
                        acc_out.at[pl.ds(c * N_NODES + s * RPT, RPT)])

    @pl.when(s == 15)
    def _drain_b():
        pltpu.sync_copy(acc_sh.at[pl.ds(15 * RPT, RPT_LAST)],
                        acc_out.at[pl.ds(c * N_NODES + 15 * RPT, RPT_LAST)])


def _matmul_body(dega_ref, degb_ref, x_ref, w_ref, g_ref):
    deg = dega_ref[...] + degb_ref[...] + 1.0
    dis = lax.rsqrt(deg)
    h = jnp.dot(x_ref[...], w_ref[...], preferred_element_type=jnp.float32)
    g_ref[...] = h * dis


def _epilogue_body(acca_ref, accb_ref, g_ref, dega_ref, degb_ref, x_ref,
                   b_ref, gam_ref, bet_ref, o_ref):
    deg = dega_ref[...] + degb_ref[...] + 1.0
    dis = lax.rsqrt(deg)
    ssum = acca_ref[...] + accb_ref[...] + g_ref[...]
    pre = ssum * dis + b_ref[...]
    bn = pre * (gam_ref[...] * BN_SCALE) + bet_ref[...]
    o_ref[...] = jnp.maximum(bn, 0.0) + x_ref[...]


def kernel(x, edge_index, W, b, gamma, beta):
    n_pad = E_PAD - N_EDGES
    src = edge_index[0].astype(jnp.int32)
    dst = edge_index[1].astype(jnp.int32)
    pad_src = (jnp.arange(n_pad, dtype=jnp.int32) * 13) % N_NODES
    pad_dst = N_NODES + (jnp.arange(n_pad, dtype=jnp.int32) % (N_ACC - N_NODES))
    src2 = jnp.concatenate([src, pad_src]).reshape(NW * NCHUNKS, CHUNK)
    dst2 = jnp.concatenate([dst, pad_dst]).reshape(NW * NCHUNKS, CHUNK)
    zeros1 = jnp.zeros((N_ACC,), jnp.float32)
    zrows = jnp.zeros((RPT_LAST, DIMS), jnp.float32)

    deg2 = _deg_kernel(dst2, zeros1)
    dega = deg2[0:N_NODES].reshape(N_NODES, 1)
    degb = deg2[N_NODES:2 * N_NODES].reshape(N_NODES, 1)

    grid = 10
    br = N_NODES // grid
    g = pl.pallas_call(
        _matmul_body,
        grid=(grid,),
        in_specs=[
            pl.BlockSpec((br, 1), lambda i: (i, 0)),
            pl.BlockSpec((br, 1), lambda i: (i, 0)),
            pl.BlockSpec((br, DIMS), lambda i: (i, 0)),
            pl.BlockSpec((DIMS, DIMS), lambda i: (0, 0)),
        ],
        out_specs=pl.BlockSpec((br, DIMS), lambda i: (i, 0)),
        out_shape=jax.ShapeDtypeStruct((N_NODES, DIMS), jnp.float32),
    )(dega, degb, x, W)

    acc = _scatter_kernel(src2, dst2, g, zrows)

    out = pl.pallas_call(
        _epilogue_body,
        grid=(grid,),
        in_specs=[
            pl.BlockSpec((br, DIMS), lambda i: (i, 0)),
            pl.BlockSpec((br, DIMS), lambda i: (i + grid, 0)),
            pl.BlockSpec((br, DIMS), lambda i: (i, 0)),
            pl.BlockSpec((br, 1), lambda i: (i, 0)),
            pl.BlockSpec((br, 1), lambda i: (i, 0)),
            pl.BlockSpec((br, DIMS), lambda i: (i, 0)),
            pl.BlockSpec((1, DIMS), lambda i: (0, 0)),
            pl.BlockSpec((1, DIMS), lambda i: (0, 0)),
            pl.BlockSpec((1, DIMS), lambda i: (0, 0)),
        ],
        out_specs=pl.BlockSpec((br, DIMS), lambda i: (i, 0)),
        out_shape=jax.ShapeDtypeStruct((N_NODES, DIMS), jnp.float32),
    )(acc, acc, g, dega, degb, x,
      b.reshape(1, DIMS), gamma.reshape(1, DIMS), beta.reshape(1, DIMS))
    return out


# speedup vs baseline: 1.1008x; 1.1008x over previous
import functools
import math

import jax
import jax.numpy as jnp
from jax import lax
from jax.experimental import pallas as pl
from jax.experimental.pallas import tpu as pltpu
from jax.experimental.pallas import tpu_sc as plsc

N_NODES = 10000
N_EDGES = 320000
DIMS = 128
NC = 2
NS = 16
NW = NC * NS
CHUNK = 80
NCHUNKS = 128
EPW = NCHUNKS * CHUNK
E_PAD = NW * EPW
N_ACC = 10240
NB = 3
NSUP = (NCHUNKS + NB - 1) // NB
DEG_WIN = 16
RPT = 624
RPT_LAST = N_NODES - 15 * RPT
BN_SCALE = 1.0 / math.sqrt(1.0 + 1e-5)

_mesh = plsc.VectorSubcoreMesh(core_axis_name="c", subcore_axis_name="s")


@functools.partial(
    pl.kernel,
    mesh=_mesh,
    out_type=jax.ShapeDtypeStruct((NC * N_NODES,), jnp.float32),
    scratch_types=[
        pltpu.VMEM((NCHUNKS, CHUNK), jnp.int32),
        pltpu.VMEM((CHUNK,), jnp.float32),
        pltpu.VMEM((N_ACC,), jnp.float32),
        pltpu.VMEM_SHARED((N_ACC,), jnp.float32),
        pltpu.SemaphoreType.DMA,
    ],
)
def _deg_kernel(dst_hbm, zeros_hbm, deg_out, dst_all, ones_v, stage_v,
                deg_sh, sem):
    c = lax.axis_index("c")
    s = lax.axis_index("s")
    w = c * NS + s
    pltpu.sync_copy(dst_hbm.at[pl.ds(w * NCHUNKS, NCHUNKS)], dst_all)
    for j in range(CHUNK // 16):
        ones_v[pl.ds(j * 16, 16)] = jnp.full((16,), 1.0, dtype=jnp.float32)

    @pl.when(s == 0)
    def _init():
        pltpu.sync_copy(zeros_hbm, stage_v)
        pltpu.sync_copy(stage_v, deg_sh)

    plsc.subcore_barrier()

    def body(i, carry):
        @pl.when(i >= DEG_WIN)
        def _throttle():
            pltpu.make_async_copy(ones_v, deg_sh.at[dst_all.at[0]], sem).wait()

        pltpu.async_copy(ones_v, deg_sh.at[dst_all.at[i]], sem, add=True)
        return carry

    lax.fori_loop(0, NCHUNKS, body, 0)

    def drain(i, carry):
        pltpu.make_async_copy(ones_v, deg_sh.at[dst_all.at[0]], sem).wait()
        return carry

    lax.fori_loop(0, DEG_WIN, drain, 0)
    plsc.subcore_barrier()

    @pl.when(s == 0)
    def _drain():
        pltpu.sync_copy(deg_sh.at[pl.ds(0, N_NODES)], stage_v.at[pl.ds(0, N_NODES)])
        pltpu.sync_copy(stage_v.at[pl.ds(0, N_NODES)],
                        deg_out.at[pl.ds(c * N_NODES, N_NODES)])


@functools.partial(
    pl.kernel,
    mesh=_mesh,
    out_type=jax.ShapeDtypeStruct((NC * N_NODES, DIMS), jnp.float32),
    scratch_types=[
        pltpu.VMEM((NCHUNKS, CHUNK), jnp.int32),
        pltpu.VMEM((NB, CHUNK), jnp.int32),
        pltpu.VMEM((NB, CHUNK, DIMS), jnp.float32),
        pltpu.VMEM_SHARED((N_ACC, DIMS), jnp.float32),
    ] + [pltpu.SemaphoreType.DMA] * (3 * NB),
)
def _scatter_kernel(src_hbm, dst_hbm, g_hbm, zrows_hbm, acc_out,
                    src_all, dst_r, rows_v, acc_sh, *sems):
    sem_g = sems[0:NB]
    sem_d = sems[NB:2 * NB]
    sem_s = sems[2 * NB:3 * NB]
    c = lax.axis_index("c")
    s = lax.axis_index("s")
    w = c * NS + s
    pltpu.sync_copy(src_hbm.at[pl.ds(w * NCHUNKS, NCHUNKS)], src_all)

    @pl.when(s < 15)
    def _init_a():
        pltpu.sync_copy(zrows_hbm.at[pl.ds(0, RPT)],
                        acc_sh.at[pl.ds(s * RPT, RPT)])

    @pl.when(s == 15)
    def _init_b():
        pltpu.sync_copy(zrows_hbm, acc_sh.at[pl.ds(15 * RPT, RPT_LAST)])

    plsc.subcore_barrier()

    for p in range(NB):
        pltpu.async_copy(g_hbm.at[src_all.at[p]], rows_v.at[p], sem_g[p])
        pltpu.async_copy(dst_hbm.at[w * NCHUNKS + p], dst_r.at[p], sem_d[p])

    def body(k, carry):
        for p in range(NB):
            i = NB * k + p

            @pl.when(i < NCHUNKS)
            def _consume(p=p, i=i):
                pltpu.make_async_copy(dst_hbm.at[0], dst_r.at[p],
                                      sem_d[p]).wait()
                pltpu.make_async_copy(g_hbm.at[src_all.at[0]], rows_v.at[p],
                                      sem_g[p]).wait()
                pltpu.async_copy(rows_v.at[p], acc_sh.at[dst_r.at[p]],
                                 sem_s[p], add=True)

        for p in range(NB):
            i = NB * k + p

            @pl.when(i < NCHUNKS)
            def _refill(p=p, i=i):
                pltpu.make_async_copy(rows_v.at[p], acc_sh.at[dst_r.at[p]],
                                      sem_s[p]).wait()

                @pl.when(i + NB < NCHUNKS)
                def _fire(p=p, i=i):
                    pltpu.async_copy(g_hbm.at[src_all.at[i + NB]],
                                     rows_v.at[p], sem_g[p])
                    pltpu.async_copy(dst_hbm.at[w * NCHUNKS + i + NB],
                                     dst_r.at[p], sem_d[p])

        return carry

    lax.fori_loop(0, NSUP, body, 0)
    plsc.subcore_barrier()

    @pl.when(s < 15)
    def _drain_a():
        pltpu.sync_copy(acc_sh.at[pl.ds(s * RPT, RPT)],
                        acc_out.at[pl.ds(c * N_NODES + s * RPT, RPT)])

    @pl.when(s == 15)
    def _drain_b():
        pltpu.sync_copy(acc_sh.at[pl.ds(15 * RPT, RPT_LAST)],
                        acc_out.at[pl.ds(c * N_NODES + 15 * RPT, RPT_LAST)])


def _matmul_body(dega_ref, degb_ref, x_ref, w_ref, g_ref):
    deg = dega_ref[...] + degb_ref[...] + 1.0
    dis = lax.rsqrt(deg)
    h = jnp.dot(x_ref[...], w_ref[...], preferred_element_type=jnp.float32)
    g_ref[...] = h * dis


def _epilogue_body(acca_ref, accb_ref, g_ref, dega_ref, degb_ref, x_ref,
                   b_ref, gam_ref, bet_ref, o_ref):
    deg = dega_ref[...] + degb_ref[...] + 1.0
    dis = lax.rsqrt(deg)
    ssum = acca_ref[...] + accb_ref[...] + g_ref[...]
    pre = ssum * dis + b_ref[...]
    bn = pre * (gam_ref[...] * BN_SCALE) + bet_ref[...]
    o_ref[...] = jnp.maximum(bn, 0.0) + x_ref[...]


def kernel(x, edge_index, W, b, gamma, beta):
    n_pad = E_PAD - N_EDGES
    src = edge_index[0].astype(jnp.int32)
    dst = edge_index[1].astype(jnp.int32)
    pad_src = (jnp.arange(n_pad, dtype=jnp.int32) * 13) % N_NODES
    pad_dst = N_NODES + (jnp.arange(n_pad, dtype=jnp.int32) % (N_ACC - N_NODES))
    src2 = jnp.concatenate([src, pad_src]).reshape(NW * NCHUNKS, CHUNK)
    dst2 = jnp.concatenate([dst, pad_dst]).reshape(NW * NCHUNKS, CHUNK)
    zeros1 = jnp.zeros((N_ACC,), jnp.float32)
    zrows = jnp.zeros((RPT_LAST, DIMS), jnp.float32)

    deg2 = _deg_kernel(dst2, zeros1)
    dega = deg2[0:N_NODES].reshape(N_NODES, 1)
    degb = deg2[N_NODES:2 * N_NODES].reshape(N_NODES, 1)

    grid = 10
    br = N_NODES // grid
    g = pl.pallas_call(
        _matmul_body,
        grid=(grid,),
        in_specs=[
            pl.BlockSpec((br, 1), lambda i: (i, 0)),
            pl.BlockSpec((br, 1), lambda i: (i, 0)),
            pl.BlockSpec((br, DIMS), lambda i: (i, 0)),
            pl.BlockSpec((DIMS, DIMS), lambda i: (0, 0)),
        ],
        out_specs=pl.BlockSpec((br, DIMS), lambda i: (i, 0)),
        out_shape=jax.ShapeDtypeStruct((N_NODES, DIMS), jnp.float32),
    )(dega, degb, x, W)

    acc = _scatter_kernel(src2, dst2, g, zrows)

    out = pl.pallas_call(
        _epilogue_body,
        grid=(grid,),
        in_specs=[
            pl.BlockSpec((br, DIMS), lambda i: (i, 0)),
            pl.BlockSpec((br, DIMS), lambda i: (i + grid, 0)),
            pl.BlockSpec((br, DIMS), lambda i: (i, 0)),
            pl.BlockSpec((br, 1), lambda i: (i, 0)),
            pl.BlockSpec((br, 1), lambda i: (i, 0)),
            pl.BlockSpec((br, DIMS), lambda i: (i, 0)),
            pl.BlockSpec((1, DIMS), lambda i: (0, 0)),
            pl.BlockSpec((1, DIMS), lambda i: (0, 0)),
            pl.BlockSpec((1, DIMS), lambda i: (0, 0)),
        ],
        out_specs=pl.BlockSpec((br, DIMS), lambda i: (i, 0)),
        out_shape=jax.ShapeDtypeStruct((N_NODES, DIMS), jnp.float32),
    )(acc, acc, g, dega, degb, x,
      b.reshape(1, DIMS), gamma.reshape(1, DIMS), beta.reshape(1, DIMS))
    return out
